# Initial kernel scaffold; baseline (speedup 1.0000x reference)
import jax
import jax.numpy as jnp
from jax.experimental import pallas as pl


def kernel(x, edge_index, W1, b1, W2, b2):
    raise NotImplementedError("write your pallas kernel here")


# speedup vs baseline: 8.6437x; 8.6437x over previous
import functools

import jax
import jax.numpy as jnp
from jax import lax
from jax.experimental import pallas as pl
from jax.experimental.pallas import tpu as pltpu
from jax.experimental.pallas import tpu_sc as plsc

N = 10000
E = 320000
D = 128

NC = 2
NS = 16
NW = NC * NS
CHUNK = 128
CPT = 80
EPT = CPT * CHUNK
EP = NW * EPT
RPT = 632
NPAD = RPT * NS

_mesh = functools.partial(
    plsc.VectorSubcoreMesh, core_axis_name="c", subcore_axis_name="s")


@functools.partial(
    pl.kernel,
    mesh=_mesh(),
    out_type=jax.ShapeDtypeStruct((NC, NPAD, D), jnp.float32),
    scratch_types=[
        pltpu.VMEM((CPT, CHUNK), jnp.int32),
        pltpu.VMEM((CHUNK, D), jnp.float32),
        pltpu.VMEM_SHARED((NPAD, D), jnp.float32),
    ],
)
def _deg_kernel(dst_hbm, ones_hbm, zeros_hbm, out_hbm, dst_v, ones_v, dacc):
    cid = lax.axis_index("c")
    sid = lax.axis_index("s")
    wid = cid * NS + sid
    pltpu.sync_copy(zeros_hbm, dacc.at[pl.ds(sid * RPT, RPT)])
    pltpu.sync_copy(dst_hbm.at[wid], dst_v)
    pltpu.sync_copy(ones_hbm, ones_v)
    plsc.subcore_barrier()

    def body(j, carry):
        pltpu.sync_copy(ones_v, dacc.at[dst_v.at[j]], add=True)
        return carry

    lax.fori_loop(0, CPT, body, 0)
    plsc.subcore_barrier()
    pltpu.sync_copy(dacc.at[pl.ds(sid * RPT, RPT)],
                    out_hbm.at[cid].at[pl.ds(sid * RPT, RPT)])


@functools.partial(
    pl.kernel,
    mesh=_mesh(),
    out_type=jax.ShapeDtypeStruct((NC, NPAD, D), jnp.float32),
    scratch_types=[
        pltpu.VMEM((CPT, CHUNK), jnp.int32),
        pltpu.VMEM((CPT, CHUNK), jnp.int32),
        pltpu.VMEM((CHUNK, D), jnp.float32),
        pltpu.VMEM_SHARED((NPAD, D), jnp.float32),
        pltpu.SemaphoreType.DMA,
    ],
)
def _agg_kernel(g_hbm, src_hbm, dst_hbm, zeros_hbm, out_hbm,
                src_v, dst_v, buf, acc, sem):
    cid = lax.axis_index("c")
    sid = lax.axis_index("s")
    wid = cid * NS + sid
    pltpu.sync_copy(zeros_hbm, acc.at[pl.ds(sid * RPT, RPT)])
    pltpu.sync_copy(src_hbm.at[wid], src_v)
    pltpu.sync_copy(dst_hbm.at[wid], dst_v)
    plsc.subcore_barrier()

    def body(j, carry):
        pltpu.async_copy(g_hbm.at[src_v.at[j]], buf, sem).wait()
        pltpu.sync_copy(buf, acc.at[dst_v.at[j]], add=True)
        return carry

    lax.fori_loop(0, CPT, body, 0)
    plsc.subcore_barrier()
    pltpu.sync_copy(acc.at[pl.ds(sid * RPT, RPT)],
                    out_hbm.at[cid].at[pl.ds(sid * RPT, RPT)])


_BR = 1000


def _stage_a_body(d0_ref, d1_ref, x_ref, w_ref, g_ref, dinv_ref):
    deg = jnp.sum(d0_ref[...] + d1_ref[...], axis=1) * (1.0 / D) + 1.0
    dinv = lax.rsqrt(deg)[:, None]
    h = jnp.dot(x_ref[...], w_ref[...], preferred_element_type=jnp.float32)
    g_ref[...] = h * dinv
    dinv_ref[...] = dinv


def _stage_b_body(dinv_ref, p0_ref, p1_ref, g_ref, b_ref, w_ref, out_ref):
    dinv = dinv_ref[...]
    agg = p0_ref[...] + p1_ref[...] + g_ref[...]
    y = jnp.maximum(agg * dinv + b_ref[...], 0.0)
    h = jnp.dot(y, w_ref[...], preferred_element_type=jnp.float32)
    out_ref[...] = h * dinv


def _stage_c_body(dinv_ref, p0_ref, p1_ref, g_ref, b_ref, out_ref):
    agg = p0_ref[...] + p1_ref[...] + g_ref[...]
    out_ref[...] = agg * dinv_ref[...] + b_ref[...]


_row_spec = pl.BlockSpec((_BR, D), lambda i: (i, 0))
_w_spec = pl.BlockSpec((D, D), lambda i: (0, 0))
_b_spec = pl.BlockSpec((1, D), lambda i: (0, 0))
_dinv_spec = pl.BlockSpec((_BR, 1), lambda i: (i, 0))
_grid = (N // _BR,)
_f32_out = jax.ShapeDtypeStruct((N, D), jnp.float32)

_stage_a = pl.pallas_call(
    _stage_a_body,
    grid=_grid,
    in_specs=[_row_spec, _row_spec, _row_spec, _w_spec],
    out_specs=[_row_spec, _dinv_spec],
    out_shape=[_f32_out, jax.ShapeDtypeStruct((N, 1), jnp.float32)],
)

_stage_b = pl.pallas_call(
    _stage_b_body,
    grid=_grid,
    in_specs=[_dinv_spec, _row_spec, _row_spec, _row_spec, _b_spec, _w_spec],
    out_specs=_row_spec,
    out_shape=_f32_out,
)

_stage_c = pl.pallas_call(
    _stage_c_body,
    grid=_grid,
    in_specs=[_dinv_spec, _row_spec, _row_spec, _row_spec, _b_spec],
    out_specs=_row_spec,
    out_shape=_f32_out,
)


def kernel(x, edge_index, W1, b1, W2, b2):
    src = edge_index[0]
    dst = edge_index[1]
    pad = EP - E
    srcp = jnp.concatenate([src, jnp.zeros((pad,), jnp.int32)])
    dstp = jnp.concatenate([dst, jnp.full((pad,), N, jnp.int32)])
    src_t = srcp.reshape(NW, CPT, CHUNK)
    dst_t = dstp.reshape(NW, CPT, CHUNK)

    ones_deg = jnp.ones((CHUNK, D), jnp.float32)
    zeros_acc = jnp.zeros((RPT, D), jnp.float32)

    degp = _deg_kernel(dst_t, ones_deg, zeros_acc)

    g1, dinv = _stage_a(degp[0, :N], degp[1, :N], x, W1)
    agg1 = _agg_kernel(g1, src_t, dst_t, zeros_acc)
    g2 = _stage_b(dinv, agg1[0, :N], agg1[1, :N], g1, b1[None, :], W2)
    agg2 = _agg_kernel(g2, src_t, dst_t, zeros_acc)
    return _stage_c(dinv, agg2[0, :N], agg2[1, :N], g2, b2[None, :])


# speedup vs baseline: 23.0562x; 2.6674x over previous
import functools

import jax
import jax.numpy as jnp
from jax import lax
from jax.experimental import pallas as pl
from jax.experimental.pallas import tpu as pltpu
from jax.experimental.pallas import tpu_sc as plsc

N = 10000
E = 320000
D = 128

NC = 2
NS = 16
NW = NC * NS
CHUNK = 128
CPT = 80
EPT = CPT * CHUNK
EP = NW * EPT
RPT = 632
NPAD = RPT * NS
KB = 2
HCPT = CPT // 2

_mesh = functools.partial(
    plsc.VectorSubcoreMesh, core_axis_name="c", subcore_axis_name="s")


@functools.partial(
    pl.kernel,
    mesh=_mesh(),
    out_type=jax.ShapeDtypeStruct((NC, NPAD, D), jnp.float32),
    scratch_types=[
        pltpu.VMEM((CPT, CHUNK), jnp.int32),
        pltpu.VMEM((CHUNK, D), jnp.float32),
        pltpu.VMEM_SHARED((NPAD, D), jnp.float32),
    ],
)
def _deg_kernel(dst_hbm, ones_hbm, zeros_hbm, out_hbm, dst_v, ones_v, dacc):
    cid = lax.axis_index("c")
    sid = lax.axis_index("s")
    wid = cid * NS + sid
    pltpu.sync_copy(zeros_hbm, dacc.at[pl.ds(sid * RPT, RPT)])
    pltpu.sync_copy(dst_hbm.at[wid], dst_v)
    pltpu.sync_copy(ones_hbm, ones_v)
    plsc.subcore_barrier()

    def body(j, carry):
        pltpu.sync_copy(ones_v, dacc.at[dst_v.at[j]], add=True)
        return carry

    lax.fori_loop(0, CPT, body, 0)
    plsc.subcore_barrier()
    pltpu.sync_copy(dacc.at[pl.ds(sid * RPT, RPT)],
                    out_hbm.at[cid].at[pl.ds(sid * RPT, RPT)])


@functools.partial(
    pl.kernel,
    mesh=_mesh(),
    out_type=jax.ShapeDtypeStruct((NC, NPAD, D), jnp.float32),
    scratch_types=[
        pltpu.VMEM((HCPT, CHUNK), jnp.int32),
        pltpu.VMEM((HCPT, CHUNK), jnp.int32),
        pltpu.VMEM((KB, CHUNK, D), jnp.float32),
        pltpu.VMEM_SHARED((NPAD, D), jnp.float32),
        pltpu.SemaphoreType.DMA,
    ],
)
def _agg_kernel(g_hbm, src_hbm, dst_hbm, zeros_hbm, out_hbm,
                src_v, dst_v, bufs, acc, sem):
    cid = lax.axis_index("c")
    sid = lax.axis_index("s")
    wid = cid * NS + sid
    pltpu.sync_copy(zeros_hbm, acc.at[pl.ds(sid * RPT, RPT)])
    plsc.subcore_barrier()

    def body(i, carry):
        j0 = i * KB
        handles = [
            pltpu.async_copy(g_hbm.at[src_v.at[j0 + b]], bufs.at[b], sem)
            for b in range(KB)
        ]
        for b in range(KB):
            handles[b].wait()
            pltpu.sync_copy(bufs.at[b], acc.at[dst_v.at[j0 + b]], add=True)
        return carry

    for h in range(2):
        pltpu.sync_copy(src_hbm.at[wid].at[pl.ds(h * HCPT, HCPT)], src_v)
        pltpu.sync_copy(dst_hbm.at[wid].at[pl.ds(h * HCPT, HCPT)], dst_v)
        lax.fori_loop(0, HCPT // KB, body, 0)
    plsc.subcore_barrier()
    pltpu.sync_copy(acc.at[pl.ds(sid * RPT, RPT)],
                    out_hbm.at[cid].at[pl.ds(sid * RPT, RPT)])


_BR = 1000


def _stage_a_body(d0_ref, d1_ref, x_ref, w_ref, g_ref, dinv_ref):
    deg = jnp.sum(d0_ref[...] + d1_ref[...], axis=1) * (1.0 / D) + 1.0
    dinv = lax.rsqrt(deg)[:, None]
    h = jnp.dot(x_ref[...], w_ref[...], preferred_element_type=jnp.float32)
    g_ref[...] = h * dinv
    dinv_ref[...] = dinv


def _stage_b_body(dinv_ref, p0_ref, p1_ref, g_ref, b_ref, w_ref, out_ref):
    dinv = dinv_ref[...]
    agg = p0_ref[...] + p1_ref[...] + g_ref[...]
    y = jnp.maximum(agg * dinv + b_ref[...], 0.0)
    h = jnp.dot(y, w_ref[...], preferred_element_type=jnp.float32)
    out_ref[...] = h * dinv


def _stage_c_body(dinv_ref, p0_ref, p1_ref, g_ref, b_ref, out_ref):
    agg = p0_ref[...] + p1_ref[...] + g_ref[...]
    out_ref[...] = agg * dinv_ref[...] + b_ref[...]


_row_spec = pl.BlockSpec((_BR, D), lambda i: (i, 0))
_w_spec = pl.BlockSpec((D, D), lambda i: (0, 0))
_b_spec = pl.BlockSpec((1, D), lambda i: (0, 0))
_dinv_spec = pl.BlockSpec((_BR, 1), lambda i: (i, 0))
_grid = (N // _BR,)
_f32_out = jax.ShapeDtypeStruct((N, D), jnp.float32)

_stage_a = pl.pallas_call(
    _stage_a_body,
    grid=_grid,
    in_specs=[_row_spec, _row_spec, _row_spec, _w_spec],
    out_specs=[_row_spec, _dinv_spec],
    out_shape=[_f32_out, jax.ShapeDtypeStruct((N, 1), jnp.float32)],
)

_stage_b = pl.pallas_call(
    _stage_b_body,
    grid=_grid,
    in_specs=[_dinv_spec, _row_spec, _row_spec, _row_spec, _b_spec, _w_spec],
    out_specs=_row_spec,
    out_shape=_f32_out,
)

_stage_c = pl.pallas_call(
    _stage_c_body,
    grid=_grid,
    in_specs=[_dinv_spec, _row_spec, _row_spec, _row_spec, _b_spec],
    out_specs=_row_spec,
    out_shape=_f32_out,
)


def kernel(x, edge_index, W1, b1, W2, b2):
    src = edge_index[0]
    dst = edge_index[1]
    pad = EP - E
    r = jnp.arange(pad, dtype=jnp.int32)
    srcp = jnp.concatenate([src, r % N])
    dstp = jnp.concatenate([dst, N + r % (NPAD - N)])
    src_t = srcp.reshape(NW, CPT, CHUNK)
    dst_t = dstp.reshape(NW, CPT, CHUNK)

    ones_deg = jnp.ones((CHUNK, D), jnp.float32)
    zeros_acc = jnp.zeros((RPT, D), jnp.float32)

    degp = _deg_kernel(dst_t, ones_deg, zeros_acc)

    g1, dinv = _stage_a(degp[0, :N], degp[1, :N], x, W1)
    agg1 = _agg_kernel(g1, src_t, dst_t, zeros_acc)
    g2 = _stage_b(dinv, agg1[0, :N], agg1[1, :N], g1, b1[None, :], W2)
    agg2 = _agg_kernel(g2, src_t, dst_t, zeros_acc)
    return _stage_c(dinv, agg2[0, :N], agg2[1, :N], g2, b2[None, :])


# speedup vs baseline: 25.5197x; 1.1068x over previous
import functools

import jax
import jax.numpy as jnp
from jax import lax
from jax.experimental import pallas as pl
from jax.experimental.pallas import tpu as pltpu
from jax.experimental.pallas import tpu_sc as plsc

N = 10000
E = 320000
D = 128

NC = 2
NS = 16
NW = NC * NS
CHUNK = 128
CPT = 80
EPT = CPT * CHUNK
EP = NW * EPT
RPT = 632
NPAD = RPT * NS
KB = 2
HCPT = CPT // 2

_mesh = functools.partial(
    plsc.VectorSubcoreMesh, core_axis_name="c", subcore_axis_name="s")


DW = D

@functools.partial(
    pl.kernel,
    mesh=_mesh(),
    out_type=jax.ShapeDtypeStruct((NC, NPAD, DW), jnp.float32),
    scratch_types=[
        pltpu.VMEM((CPT, CHUNK), jnp.int32),
        pltpu.VMEM((CHUNK, DW), jnp.float32),
        pltpu.VMEM_SHARED((NPAD, DW), jnp.float32),
    ],
)
def _deg_kernel(dst_hbm, ones_hbm, zeros_hbm, out_hbm, dst_v, ones_v, dacc):
    cid = lax.axis_index("c")
    sid = lax.axis_index("s")
    wid = cid * NS + sid
    pltpu.sync_copy(zeros_hbm, dacc.at[pl.ds(sid * RPT, RPT)])
    pltpu.sync_copy(dst_hbm.at[wid], dst_v)
    pltpu.sync_copy(ones_hbm, ones_v)
    plsc.subcore_barrier()

    def body(j, carry):
        pltpu.sync_copy(ones_v, dacc.at[dst_v.at[j]], add=True)
        return carry

    lax.fori_loop(0, CPT, body, 0)
    plsc.subcore_barrier()
    pltpu.sync_copy(dacc.at[pl.ds(sid * RPT, RPT)],
                    out_hbm.at[cid].at[pl.ds(sid * RPT, RPT)])


@functools.partial(
    pl.kernel,
    mesh=_mesh(),
    out_type=jax.ShapeDtypeStruct((NC, NPAD, D), jnp.float32),
    scratch_types=[
        pltpu.VMEM((HCPT, CHUNK), jnp.int32),
        pltpu.VMEM((HCPT, CHUNK), jnp.int32),
        pltpu.VMEM((KB, CHUNK, D), jnp.float32),
        pltpu.VMEM_SHARED((NPAD, D), jnp.float32),
        pltpu.SemaphoreType.DMA,
    ],
)
def _agg_kernel(g_hbm, src_hbm, dst_hbm, zeros_hbm, out_hbm,
                src_v, dst_v, bufs, acc, sem):
    cid = lax.axis_index("c")
    sid = lax.axis_index("s")
    wid = cid * NS + sid
    pltpu.sync_copy(zeros_hbm, acc.at[pl.ds(sid * RPT, RPT)])
    plsc.subcore_barrier()

    def body(i, carry):
        for b in range(KB):
            j = KB * i + b
            pltpu.make_async_copy(
                g_hbm.at[pl.ds(0, CHUNK)], bufs.at[b], sem).wait()
            pltpu.sync_copy(bufs.at[b], acc.at[dst_v.at[j]], add=True)
            jn = jnp.minimum(j + KB, HCPT - 1)
            pltpu.async_copy(g_hbm.at[src_v.at[jn]], bufs.at[b], sem)
        return carry

    for h in range(2):
        pltpu.sync_copy(src_hbm.at[wid].at[pl.ds(h * HCPT, HCPT)], src_v)
        pltpu.sync_copy(dst_hbm.at[wid].at[pl.ds(h * HCPT, HCPT)], dst_v)
        for b in range(KB):
            pltpu.async_copy(g_hbm.at[src_v.at[b]], bufs.at[b], sem)
        lax.fori_loop(0, HCPT // KB, body, 0)
        for b in range(KB):
            pltpu.make_async_copy(
                g_hbm.at[pl.ds(0, CHUNK)], bufs.at[b], sem).wait()
    plsc.subcore_barrier()
    pltpu.sync_copy(acc.at[pl.ds(sid * RPT, RPT)],
                    out_hbm.at[cid].at[pl.ds(sid * RPT, RPT)])


_BR = 1000


def _stage_a_body(d0_ref, d1_ref, x_ref, w_ref, g_ref, dinv_ref):
    deg = jnp.sum(d0_ref[...] + d1_ref[...], axis=1) * (1.0 / DW) + 1.0
    dinv = lax.rsqrt(deg)[:, None]
    h = jnp.dot(x_ref[...], w_ref[...], preferred_element_type=jnp.float32)
    g_ref[...] = h * dinv
    dinv_ref[...] = dinv


def _stage_b_body(dinv_ref, p0_ref, p1_ref, g_ref, b_ref, w_ref, out_ref):
    dinv = dinv_ref[...]
    agg = p0_ref[...] + p1_ref[...] + g_ref[...]
    y = jnp.maximum(agg * dinv + b_ref[...], 0.0)
    h = jnp.dot(y, w_ref[...], preferred_element_type=jnp.float32)
    out_ref[...] = h * dinv


def _stage_c_body(dinv_ref, p0_ref, p1_ref, g_ref, b_ref, out_ref):
    agg = p0_ref[...] + p1_ref[...] + g_ref[...]
    out_ref[...] = agg * dinv_ref[...] + b_ref[...]


_row_spec = pl.BlockSpec((_BR, D), lambda i: (i, 0))
_deg_spec = pl.BlockSpec((_BR, DW), lambda i: (i, 0))
_w_spec = pl.BlockSpec((D, D), lambda i: (0, 0))
_b_spec = pl.BlockSpec((1, D), lambda i: (0, 0))
_dinv_spec = pl.BlockSpec((_BR, 1), lambda i: (i, 0))
_grid = (N // _BR,)
_f32_out = jax.ShapeDtypeStruct((N, D), jnp.float32)

_stage_a = pl.pallas_call(
    _stage_a_body,
    grid=_grid,
    in_specs=[_deg_spec, _deg_spec, _row_spec, _w_spec],
    out_specs=[_row_spec, _dinv_spec],
    out_shape=[_f32_out, jax.ShapeDtypeStruct((N, 1), jnp.float32)],
)

_stage_b = pl.pallas_call(
    _stage_b_body,
    grid=_grid,
    in_specs=[_dinv_spec, _row_spec, _row_spec, _row_spec, _b_spec, _w_spec],
    out_specs=_row_spec,
    out_shape=_f32_out,
)

_stage_c = pl.pallas_call(
    _stage_c_body,
    grid=_grid,
    in_specs=[_dinv_spec, _row_spec, _row_spec, _row_spec, _b_spec],
    out_specs=_row_spec,
    out_shape=_f32_out,
)


def kernel(x, edge_index, W1, b1, W2, b2):
    src = edge_index[0]
    dst = edge_index[1]
    pad = EP - E
    r = jnp.arange(pad, dtype=jnp.int32)
    srcp = jnp.concatenate([src, r % N])
    dstp = jnp.concatenate([dst, N + r % (NPAD - N)])
    src_t = srcp.reshape(NW, CPT, CHUNK)
    dst_t = dstp.reshape(NW, CPT, CHUNK)

    ones_deg = jnp.ones((CHUNK, DW), jnp.float32)
    zeros_deg = jnp.zeros((RPT, DW), jnp.float32)
    zeros_acc = jnp.zeros((RPT, D), jnp.float32)

    degp = _deg_kernel(dst_t, ones_deg, zeros_deg)

    g1, dinv = _stage_a(degp[0, :N], degp[1, :N], x, W1)
    agg1 = _agg_kernel(g1, src_t, dst_t, zeros_acc)
    g2 = _stage_b(dinv, agg1[0, :N], agg1[1, :N], g1, b1[None, :], W2)
    agg2 = _agg_kernel(g2, src_t, dst_t, zeros_acc)
    return _stage_c(dinv, agg2[0, :N], agg2[1, :N], g2, b2[None, :])
